# baseline (device time: 457070 ns/iter reference)
import jax
import jax.numpy as jnp
from jax import lax
from jax.experimental import pallas as pl
from jax.experimental.pallas import tpu as pltpu

N_DEV = 16


def _mm(a, b):
    return lax.dot_general(
        a.astype(jnp.bfloat16),
        b.astype(jnp.bfloat16),
        (((1,), (0,)), ((), ())),
        preferred_element_type=jnp.float32,
    )


def kernel(x, w_mat, scale_x, scale_w):
    m_rows, n = w_mat.shape
    assert x.shape == (N_DEV * m_rows, m_rows)
    x = x.astype(jnp.float8_e5m2)
    w_mat = w_mat.astype(jnp.float8_e5m2)

    half = n // 2

    def body(x_ref, w_ref, sx_ref, sw_ref, out_ref,
             xg_ref, wg_ref,
             a2a_send_sems, a2a_recv_sems,
             ring_send0, ring_send1, ring_recv0, ring_recv1):
        my = lax.axis_index("i")
        right = lax.rem(my + 1, N_DEV)

        barrier = pltpu.get_barrier_semaphore()
        for k in range(1, N_DEV):
            pl.semaphore_signal(
                barrier, inc=1,
                device_id=(lax.rem(my + k, N_DEV),),
                device_id_type=pl.DeviceIdType.MESH,
            )
        pl.semaphore_wait(barrier, N_DEV - 1)

        wg_ref[0, :, :] = w_ref[:, :]
        xg_ref[0, :, :] = x_ref[pl.ds(my * m_rows, m_rows), :]

        a2a = []
        for dj in range(1, N_DEV):
            dst = lax.rem(my + dj, N_DEV)
            r = pltpu.make_async_remote_copy(
                src_ref=x_ref.at[pl.ds(dst * m_rows, m_rows), :],
                dst_ref=xg_ref.at[dj],
                send_sem=a2a_send_sems.at[dj],
                recv_sem=a2a_recv_sems.at[dj],
                device_id=(dst,),
                device_id_type=pl.DeviceIdType.MESH,
            )
            a2a.append(r)
        a2a[0].start()
        a2a[1].start()

        scale = sx_ref[0] * sw_ref[0]

        def desc(h, s):
            send_sems = ring_send0 if s == 0 else ring_send1
            recv_sems = ring_recv0 if s == 0 else ring_recv1
            return pltpu.make_async_remote_copy(
                src_ref=wg_ref.at[h, :, pl.ds(s * half, half)],
                dst_ref=wg_ref.at[h + 1, :, pl.ds(s * half, half)],
                send_sem=send_sems.at[h],
                recv_sem=recv_sems.at[h + 1],
                device_id=(right,),
                device_id_type=pl.DeviceIdType.MESH,
            )

        descs = [[desc(h, 0), desc(h, 1)] for h in range(N_DEV - 1)]
        descs[0][0].start()
        descs[0][1].start()
        out_ref[:, :] = _mm(xg_ref[0], wg_ref[0])

        for h in range(N_DEV - 1):
            if h + 2 < N_DEV - 1:
                a2a[h + 2].start()
            descs[h][0].wait_recv()
            if h < N_DEV - 2:
                descs[h + 1][0].start()
                descs[h][1].wait_recv()
                descs[h + 1][1].start()
                a2a[h].wait_recv()
                out_ref[:, :] += _mm(xg_ref[h + 1], wg_ref[h + 1])

        last = N_DEV - 1
        a2a[last - 1].wait_recv()
        acc0 = (out_ref[:, :half]
                + _mm(xg_ref[last], wg_ref[last, :, :half]))
        out_ref[:, :half] = jnp.maximum(acc0 * scale, 0.0)
        descs[last - 1][1].wait_recv()
        acc1 = (out_ref[:, half:]
                + _mm(xg_ref[last], wg_ref[last, :, half:]))
        out_ref[:, half:] = jnp.maximum(acc1 * scale, 0.0)

        for pair in descs:
            pair[0].wait_send()
            pair[1].wait_send()
        for r in a2a:
            r.wait_send()

    return pl.pallas_call(
        body,
        out_shape=jax.ShapeDtypeStruct((m_rows, n), jnp.float32),
        in_specs=[
            pl.BlockSpec(memory_space=pltpu.VMEM),
            pl.BlockSpec(memory_space=pltpu.VMEM),
            pl.BlockSpec(memory_space=pltpu.SMEM),
            pl.BlockSpec(memory_space=pltpu.SMEM),
        ],
        out_specs=pl.BlockSpec(memory_space=pltpu.VMEM),
        scratch_shapes=[
            pltpu.VMEM((N_DEV, m_rows, m_rows), x.dtype),
            pltpu.VMEM((N_DEV, m_rows, n), w_mat.dtype),
            pltpu.SemaphoreType.DMA((N_DEV,)),
            pltpu.SemaphoreType.DMA((N_DEV,)),
            pltpu.SemaphoreType.DMA((N_DEV,)),
            pltpu.SemaphoreType.DMA((N_DEV,)),
            pltpu.SemaphoreType.DMA((N_DEV,)),
            pltpu.SemaphoreType.DMA((N_DEV,)),
        ],
        compiler_params=pltpu.CompilerParams(
            collective_id=0,
            vmem_limit_bytes=56 * 1024 * 1024,
        ),
    )(x, w_mat, scale_x, scale_w)


# device time: 377283 ns/iter; 1.2115x vs baseline; 1.2115x over previous
import jax
import jax.numpy as jnp
from jax import lax
from jax.experimental import pallas as pl
from jax.experimental.pallas import tpu as pltpu

N_DEV = 16


def _mm(a, b):
    return lax.dot_general(
        a.astype(jnp.bfloat16),
        b.astype(jnp.bfloat16),
        (((1,), (0,)), ((), ())),
        preferred_element_type=jnp.float32,
    )


def kernel(x, w_mat, scale_x, scale_w):
    m_rows, n = w_mat.shape
    assert x.shape == (N_DEV * m_rows, m_rows)
    x = x.astype(jnp.float8_e5m2)
    w_mat = w_mat.astype(jnp.float8_e5m2)

    half = n // 2

    def body(x_ref, w_ref, sx_ref, sw_ref, out_ref,
             xg_ref, wg_ref,
             a2a_send_sems, a2a_recv_sems,
             ring_send0, ring_send1, ring_recv0, ring_recv1):
        my = lax.axis_index("i")
        right = lax.rem(my + 1, N_DEV)

        barrier = pltpu.get_barrier_semaphore()
        for k in range(1, N_DEV):
            pl.semaphore_signal(
                barrier, inc=1,
                device_id=(lax.rem(my + k, N_DEV),),
                device_id_type=pl.DeviceIdType.MESH,
            )
        pl.semaphore_wait(barrier, N_DEV - 1)

        wg_ref[0, :, :] = w_ref[:, :]
        xg_ref[0, :, :] = x_ref[pl.ds(my * m_rows, m_rows), :]

        a2a = []
        for dj in range(1, N_DEV):
            dst = lax.rem(my + dj, N_DEV)
            r = pltpu.make_async_remote_copy(
                src_ref=x_ref.at[pl.ds(dst * m_rows, m_rows), :],
                dst_ref=xg_ref.at[dj],
                send_sem=a2a_send_sems.at[dj],
                recv_sem=a2a_recv_sems.at[dj],
                device_id=(dst,),
                device_id_type=pl.DeviceIdType.MESH,
            )
            r.start()
            a2a.append(r)

        scale = sx_ref[0] * sw_ref[0]

        def desc(h, s):
            send_sems = ring_send0 if s == 0 else ring_send1
            recv_sems = ring_recv0 if s == 0 else ring_recv1
            return pltpu.make_async_remote_copy(
                src_ref=wg_ref.at[h, :, pl.ds(s * half, half)],
                dst_ref=wg_ref.at[h + 1, :, pl.ds(s * half, half)],
                send_sem=send_sems.at[h],
                recv_sem=recv_sems.at[h + 1],
                device_id=(right,),
                device_id_type=pl.DeviceIdType.MESH,
            )

        descs = [[desc(h, 0), desc(h, 1)] for h in range(N_DEV - 1)]
        descs[0][0].start()
        descs[0][1].start()
        out_ref[:, :] = _mm(xg_ref[0], wg_ref[0])

        for h in range(N_DEV - 1):
            descs[h][0].wait_recv()
            if h < N_DEV - 2:
                descs[h + 1][0].start()
                descs[h][1].wait_recv()
                descs[h + 1][1].start()
                a2a[h].wait_recv()
                out_ref[:, :] += _mm(xg_ref[h + 1], wg_ref[h + 1])

        last = N_DEV - 1
        a2a[last - 1].wait_recv()
        acc0 = (out_ref[:, :half]
                + _mm(xg_ref[last], wg_ref[last, :, :half]))
        out_ref[:, :half] = jnp.maximum(acc0 * scale, 0.0)
        descs[last - 1][1].wait_recv()
        acc1 = (out_ref[:, half:]
                + _mm(xg_ref[last], wg_ref[last, :, half:]))
        out_ref[:, half:] = jnp.maximum(acc1 * scale, 0.0)

        for pair in descs:
            pair[0].wait_send()
            pair[1].wait_send()
        for r in a2a:
            r.wait_send()

    return pl.pallas_call(
        body,
        out_shape=jax.ShapeDtypeStruct((m_rows, n), jnp.float32),
        in_specs=[
            pl.BlockSpec(memory_space=pltpu.VMEM),
            pl.BlockSpec(memory_space=pltpu.VMEM),
            pl.BlockSpec(memory_space=pltpu.SMEM),
            pl.BlockSpec(memory_space=pltpu.SMEM),
        ],
        out_specs=pl.BlockSpec(memory_space=pltpu.VMEM),
        scratch_shapes=[
            pltpu.VMEM((N_DEV, m_rows, m_rows), x.dtype),
            pltpu.VMEM((N_DEV, m_rows, n), w_mat.dtype),
            pltpu.SemaphoreType.DMA((N_DEV,)),
            pltpu.SemaphoreType.DMA((N_DEV,)),
            pltpu.SemaphoreType.DMA((N_DEV,)),
            pltpu.SemaphoreType.DMA((N_DEV,)),
            pltpu.SemaphoreType.DMA((N_DEV,)),
            pltpu.SemaphoreType.DMA((N_DEV,)),
        ],
        compiler_params=pltpu.CompilerParams(
            collective_id=0,
            vmem_limit_bytes=56 * 1024 * 1024,
        ),
    )(x, w_mat, scale_x, scale_w)
